# Initial kernel scaffold; baseline (speedup 1.0000x reference)
import jax
import jax.numpy as jnp
from jax.experimental import pallas as pl


def kernel(x, edge_index, W1, b1, W2, b2):
    raise NotImplementedError("write your pallas kernel here")


# speedup vs baseline: 20.4529x; 20.4529x over previous
import functools

import jax
import jax.numpy as jnp
from jax import lax
from jax.experimental import pallas as pl
from jax.experimental.pallas import tpu as pltpu
from jax.experimental.pallas import tpu_sc as plsc

N = 10000
NP = 10240
E = 320000
D_IN = 128
D_HID = 64
D_PAD = 16

NC = 2
NS = 16
NW = NC * NS
BB = 80
EPT = E // NW
NBLK = EPT // BB
RPT = NP // NS
ZR = 128


def _make_prop(F):
    mesh = plsc.VectorSubcoreMesh(core_axis_name="c", subcore_axis_name="s")

    @functools.partial(
        pl.kernel,
        out_type=(jax.ShapeDtypeStruct((NP, F), jnp.float32),
                  jax.ShapeDtypeStruct((NP, F), jnp.float32)),
        mesh=mesh,
        compiler_params=pltpu.CompilerParams(use_tc_tiling_on_sc=False),
        scratch_types=[
            pltpu.VMEM_SHARED((NP, F), jnp.float32),
            pltpu.VMEM((ZR, F), jnp.float32),
            pltpu.VMEM((NBLK, BB), jnp.int32),
            pltpu.VMEM((NBLK, BB), jnp.int32),
            pltpu.VMEM((BB, F), jnp.float32),
            pltpu.SemaphoreType.DMA,
        ],
    )
    def prop(u_hbm, src_hbm, dst_hbm, out_a, out_b, acc, zbuf, src_v, dst_v,
             rows_v, sem):
        c = lax.axis_index("c")
        s = lax.axis_index("s")
        zero16 = jnp.zeros((16,), jnp.float32)

        def zfill(i, carry):
            for jj in range(F // 16):
                zbuf[i, pl.ds(jj * 16, 16)] = zero16
            return carry

        lax.fori_loop(0, ZR, zfill, 0)
        row0 = s * RPT
        for b in range(RPT // ZR):
            pltpu.sync_copy(zbuf, acc.at[pl.ds(row0 + b * ZR, ZR)])

        tile = c * NS + s
        pltpu.sync_copy(src_hbm.at[tile], src_v)
        pltpu.sync_copy(dst_hbm.at[tile], dst_v)

        plsc.subcore_barrier()

        def body(j, carry):
            pltpu.async_copy(u_hbm.at[src_v.at[j]], rows_v, sem).wait()
            pltpu.sync_copy(rows_v, acc.at[dst_v.at[j]], add=True)
            return carry

        lax.fori_loop(0, NBLK, body, 0)

        plsc.subcore_barrier()

        @pl.when(c == 0)
        def _():
            pltpu.sync_copy(acc.at[pl.ds(row0, RPT)],
                            out_a.at[pl.ds(row0, RPT)])

        @pl.when(c == 1)
        def _():
            pltpu.sync_copy(acc.at[pl.ds(row0, RPT)],
                            out_b.at[pl.ds(row0, RPT)])

    return prop


def _make_deg():
    mesh = plsc.VectorSubcoreMesh(core_axis_name="c", subcore_axis_name="s")
    F = 16

    @functools.partial(
        pl.kernel,
        out_type=(jax.ShapeDtypeStruct((NP, F), jnp.float32),
                  jax.ShapeDtypeStruct((NP, F), jnp.float32)),
        mesh=mesh,
        compiler_params=pltpu.CompilerParams(use_tc_tiling_on_sc=False),
        scratch_types=[
            pltpu.VMEM_SHARED((NP, F), jnp.float32),
            pltpu.VMEM((ZR, F), jnp.float32),
            pltpu.VMEM((NBLK, BB), jnp.int32),
            pltpu.VMEM((BB, F), jnp.float32),
        ],
    )
    def deg(dst_hbm, out_a, out_b, acc, zbuf, dst_v, ones_v):
        c = lax.axis_index("c")
        s = lax.axis_index("s")
        zero16 = jnp.zeros((16,), jnp.float32)
        one16 = jnp.ones((16,), jnp.float32)

        def zfill(i, carry):
            zbuf[i, pl.ds(0, 16)] = zero16
            return carry

        lax.fori_loop(0, ZR, zfill, 0)

        def ofill(i, carry):
            ones_v[i, pl.ds(0, 16)] = one16
            return carry

        lax.fori_loop(0, BB, ofill, 0)

        row0 = s * RPT
        for b in range(RPT // ZR):
            pltpu.sync_copy(zbuf, acc.at[pl.ds(row0 + b * ZR, ZR)])

        tile = c * NS + s
        pltpu.sync_copy(dst_hbm.at[tile], dst_v)

        plsc.subcore_barrier()

        def body(j, carry):
            pltpu.sync_copy(ones_v, acc.at[dst_v.at[j]], add=True)
            return carry

        lax.fori_loop(0, NBLK, body, 0)

        plsc.subcore_barrier()

        @pl.when(c == 0)
        def _():
            pltpu.sync_copy(acc.at[pl.ds(row0, RPT)],
                            out_a.at[pl.ds(row0, RPT)])

        @pl.when(c == 1)
        def _():
            pltpu.sync_copy(acc.at[pl.ds(row0, RPT)],
                            out_b.at[pl.ds(row0, RPT)])

    return deg


_prop64 = _make_prop(D_HID)
_prop16 = _make_prop(D_PAD)
_deg = _make_deg()


_R = 640
_G = NP // _R


def _dinvs(dega, degb):
    deg = dega[:, 0:1] + degb[:, 0:1] + 1.0
    dinv = lax.rsqrt(deg)
    return dinv, 1.0 / deg


def _tc_mm1_body(x_ref, w_ref, dega_ref, degb_ref, t_ref, u_ref):
    t = jnp.dot(x_ref[...], w_ref[...], preferred_element_type=jnp.float32)
    dinv, _ = _dinvs(dega_ref[...], degb_ref[...])
    t_ref[...] = t
    u_ref[...] = t * dinv


def _tc_comb_body(ga_ref, gb_ref, v_ref, dega_ref, degb_ref, x_ref, u_ref):
    dinv, dinv2 = _dinvs(dega_ref[...], degb_ref[...])
    x1 = dinv * (ga_ref[...] + gb_ref[...]) + dinv2 * v_ref[...]
    x_ref[...] = x1
    u_ref[...] = dinv * x1


def _tc_mid_body(ga_ref, gb_ref, v_ref, dega_ref, degb_ref, b1_ref, w2_ref,
                 t_ref, u_ref):
    dinv, dinv2 = _dinvs(dega_ref[...], degb_ref[...])
    x2 = dinv * (ga_ref[...] + gb_ref[...]) + dinv2 * v_ref[...]
    h = jnp.maximum(x2 + b1_ref[...], 0.0)
    t2 = jnp.dot(h, w2_ref[...], preferred_element_type=jnp.float32)
    t_ref[...] = t2
    u_ref[...] = t2 * dinv


def _tc_out_body(ga_ref, gb_ref, v_ref, dega_ref, degb_ref, b2_ref, o_ref):
    dinv, dinv2 = _dinvs(dega_ref[...], degb_ref[...])
    o = dinv * (ga_ref[...] + gb_ref[...]) + dinv2 * v_ref[...] + b2_ref[...]
    col = lax.broadcasted_iota(jnp.int32, o.shape, 1)
    m = col < 6
    neg = jnp.float32(-1e30)
    mx = jnp.max(jnp.where(m, o, neg), axis=1, keepdims=True)
    ex = jnp.where(m, jnp.exp(o - mx), 0.0)
    se = jnp.sum(ex, axis=1, keepdims=True)
    o_ref[...] = (o - mx) - jnp.log(se)


def _rows(F):
    return pl.BlockSpec((_R, F), lambda i: (i, 0))


def _full(shape):
    return pl.BlockSpec(shape, lambda i: tuple(0 for _ in shape))


_f32 = jnp.float32


def _tc_mm1(x, w1, dega, degb):
    return pl.pallas_call(
        _tc_mm1_body,
        grid=(_G,),
        in_specs=[_rows(D_IN), _full((D_IN, D_HID)), _rows(16), _rows(16)],
        out_specs=[_rows(D_HID), _rows(D_HID)],
        out_shape=[jax.ShapeDtypeStruct((NP, D_HID), _f32)] * 2,
    )(x, w1, dega, degb)


def _tc_comb(ga, gb, v, dega, degb):
    F = ga.shape[1]
    return pl.pallas_call(
        _tc_comb_body,
        grid=(_G,),
        in_specs=[_rows(F), _rows(F), _rows(F), _rows(16), _rows(16)],
        out_specs=[_rows(F), _rows(F)],
        out_shape=[jax.ShapeDtypeStruct((NP, F), _f32)] * 2,
    )(ga, gb, v, dega, degb)


def _tc_mid(ga, gb, v, dega, degb, b1r, w2p):
    return pl.pallas_call(
        _tc_mid_body,
        grid=(_G,),
        in_specs=[_rows(D_HID), _rows(D_HID), _rows(D_HID), _rows(16),
                  _rows(16), _full((1, D_HID)), _full((D_HID, D_PAD))],
        out_specs=[_rows(D_PAD), _rows(D_PAD)],
        out_shape=[jax.ShapeDtypeStruct((NP, D_PAD), _f32)] * 2,
    )(ga, gb, v, dega, degb, b1r, w2p)


def _tc_out(ga, gb, v, dega, degb, b2r):
    return pl.pallas_call(
        _tc_out_body,
        grid=(_G,),
        in_specs=[_rows(D_PAD), _rows(D_PAD), _rows(D_PAD), _rows(16),
                  _rows(16), _full((1, D_PAD))],
        out_specs=_rows(D_PAD),
        out_shape=jax.ShapeDtypeStruct((NP, D_PAD), _f32),
    )(ga, gb, v, dega, degb, b2r)


def kernel(x, edge_index, W1, b1, W2, b2):
    xp = jnp.pad(x, ((0, NP - N), (0, 0)))
    src2d = edge_index[0].reshape(NW, NBLK, BB)
    dst2d = edge_index[1].reshape(NW, NBLK, BB)
    b1r = b1.reshape(1, D_HID)
    w2p = jnp.zeros((D_HID, D_PAD), _f32).at[:, :6].set(W2)
    b2r = jnp.zeros((1, D_PAD), _f32).at[0, :6].set(b2)

    dega, degb = _deg(dst2d)
    t1, u1 = _tc_mm1(xp, W1, dega, degb)
    ga, gb = _prop64(u1, src2d, dst2d)
    x1, u2 = _tc_comb(ga, gb, t1, dega, degb)
    ga, gb = _prop64(u2, src2d, dst2d)
    t2, u3 = _tc_mid(ga, gb, x1, dega, degb, b1r, w2p)
    ga, gb = _prop16(u3, src2d, dst2d)
    x3, u4 = _tc_comb(ga, gb, t2, dega, degb)
    ga, gb = _prop16(u4, src2d, dst2d)
    o = _tc_out(ga, gb, x3, dega, degb, b2r)
    return o[:N, :6]


# speedup vs baseline: 39.3610x; 1.9245x over previous
import functools

import jax
import jax.numpy as jnp
from jax import lax
from jax.experimental import pallas as pl
from jax.experimental.pallas import tpu as pltpu
from jax.experimental.pallas import tpu_sc as plsc

N = 10000
NP = 10240
E = 320000
D_IN = 128
D_HID = 64
D_PAD = 16

NC = 2
NS = 16
NW = NC * NS
BB = 80
EPT = E // NW
NBLK = EPT // BB
RPT = NP // NS
ZR = 128
NBUF = 5


def _make_prop(F):
    mesh = plsc.VectorSubcoreMesh(core_axis_name="c", subcore_axis_name="s")

    @functools.partial(
        pl.kernel,
        out_type=(jax.ShapeDtypeStruct((NP, F), jnp.float32),
                  jax.ShapeDtypeStruct((NP, F), jnp.float32)),
        mesh=mesh,
        compiler_params=pltpu.CompilerParams(use_tc_tiling_on_sc=False),
        scratch_types=[
            pltpu.VMEM_SHARED((NP, F), jnp.float32),
            pltpu.VMEM((ZR, F), jnp.float32),
            pltpu.VMEM((NBLK, BB), jnp.int32),
            pltpu.VMEM((NBLK, BB), jnp.int32),
        ]
        + [pltpu.VMEM((BB, F), jnp.float32) for _ in range(NBUF)]
        + [pltpu.SemaphoreType.DMA for _ in range(NBUF)],
    )
    def prop(u_hbm, src_hbm, dst_hbm, out_a, out_b, acc, zbuf, src_v, dst_v,
             *bufs):
        rows = bufs[:NBUF]
        sems = bufs[NBUF:]
        c = lax.axis_index("c")
        s = lax.axis_index("s")
        zero16 = jnp.zeros((16,), jnp.float32)

        def zfill(i, carry):
            for jj in range(F // 16):
                zbuf[i, pl.ds(jj * 16, 16)] = zero16
            return carry

        lax.fori_loop(0, ZR, zfill, 0)
        row0 = s * RPT
        for b in range(RPT // ZR):
            pltpu.sync_copy(zbuf, acc.at[pl.ds(row0 + b * ZR, ZR)])

        tile = c * NS + s
        pltpu.sync_copy(src_hbm.at[tile], src_v)
        pltpu.sync_copy(dst_hbm.at[tile], dst_v)

        plsc.subcore_barrier()

        for k in range(NBUF):
            pltpu.async_copy(u_hbm.at[src_v.at[k]], rows[k], sems[k])

        def body(i, carry):
            for k in range(NBUF):
                j = i * NBUF + k
                pltpu.make_async_copy(u_hbm.at[src_v.at[j]], rows[k],
                                      sems[k]).wait()
                pltpu.sync_copy(rows[k], acc.at[dst_v.at[j]], add=True)
                jn = j + NBUF

                @pl.when(jn < NBLK)
                def _():
                    pltpu.async_copy(u_hbm.at[src_v.at[jn]], rows[k],
                                     sems[k])
            return carry

        lax.fori_loop(0, NBLK // NBUF, body, 0)

        plsc.subcore_barrier()

        @pl.when(c == 0)
        def _():
            pltpu.sync_copy(acc.at[pl.ds(row0, RPT)],
                            out_a.at[pl.ds(row0, RPT)])

        @pl.when(c == 1)
        def _():
            pltpu.sync_copy(acc.at[pl.ds(row0, RPT)],
                            out_b.at[pl.ds(row0, RPT)])

    return prop


def _make_deg():
    mesh = plsc.VectorSubcoreMesh(core_axis_name="c", subcore_axis_name="s")
    F = 16

    @functools.partial(
        pl.kernel,
        out_type=(jax.ShapeDtypeStruct((NP, F), jnp.float32),
                  jax.ShapeDtypeStruct((NP, F), jnp.float32)),
        mesh=mesh,
        compiler_params=pltpu.CompilerParams(use_tc_tiling_on_sc=False),
        scratch_types=[
            pltpu.VMEM_SHARED((NP, F), jnp.float32),
            pltpu.VMEM((ZR, F), jnp.float32),
            pltpu.VMEM((NBLK, BB), jnp.int32),
            pltpu.VMEM((BB, F), jnp.float32),
        ],
    )
    def deg(dst_hbm, out_a, out_b, acc, zbuf, dst_v, ones_v):
        c = lax.axis_index("c")
        s = lax.axis_index("s")
        zero16 = jnp.zeros((16,), jnp.float32)
        one16 = jnp.ones((16,), jnp.float32)

        def zfill(i, carry):
            zbuf[i, pl.ds(0, 16)] = zero16
            return carry

        lax.fori_loop(0, ZR, zfill, 0)

        def ofill(i, carry):
            ones_v[i, pl.ds(0, 16)] = one16
            return carry

        lax.fori_loop(0, BB, ofill, 0)

        row0 = s * RPT
        for b in range(RPT // ZR):
            pltpu.sync_copy(zbuf, acc.at[pl.ds(row0 + b * ZR, ZR)])

        tile = c * NS + s
        pltpu.sync_copy(dst_hbm.at[tile], dst_v)

        plsc.subcore_barrier()

        def body(j, carry):
            pltpu.sync_copy(ones_v, acc.at[dst_v.at[j]], add=True)
            return carry

        lax.fori_loop(0, NBLK, body, 0)

        plsc.subcore_barrier()

        @pl.when(c == 0)
        def _():
            pltpu.sync_copy(acc.at[pl.ds(row0, RPT)],
                            out_a.at[pl.ds(row0, RPT)])

        @pl.when(c == 1)
        def _():
            pltpu.sync_copy(acc.at[pl.ds(row0, RPT)],
                            out_b.at[pl.ds(row0, RPT)])

    return deg


_prop64 = _make_prop(D_HID)
_prop16 = _make_prop(D_PAD)
_deg = _make_deg()


_R = 640
_G = NP // _R


def _dinvs(dega, degb):
    deg = dega[:, 0:1] + degb[:, 0:1] + 1.0
    dinv = lax.rsqrt(deg)
    return dinv, 1.0 / deg


def _tc_mm1_body(x_ref, w_ref, dega_ref, degb_ref, t_ref, u_ref):
    t = jnp.dot(x_ref[...], w_ref[...], preferred_element_type=jnp.float32)
    dinv, _ = _dinvs(dega_ref[...], degb_ref[...])
    t_ref[...] = t
    u_ref[...] = t * dinv


def _tc_comb_body(ga_ref, gb_ref, v_ref, dega_ref, degb_ref, x_ref, u_ref):
    dinv, dinv2 = _dinvs(dega_ref[...], degb_ref[...])
    x1 = dinv * (ga_ref[...] + gb_ref[...]) + dinv2 * v_ref[...]
    x_ref[...] = x1
    u_ref[...] = dinv * x1


def _tc_mid_body(ga_ref, gb_ref, v_ref, dega_ref, degb_ref, b1_ref, w2_ref,
                 t_ref, u_ref):
    dinv, dinv2 = _dinvs(dega_ref[...], degb_ref[...])
    x2 = dinv * (ga_ref[...] + gb_ref[...]) + dinv2 * v_ref[...]
    h = jnp.maximum(x2 + b1_ref[...], 0.0)
    t2 = jnp.dot(h, w2_ref[...], preferred_element_type=jnp.float32)
    t_ref[...] = t2
    u_ref[...] = t2 * dinv


def _tc_out_body(ga_ref, gb_ref, v_ref, dega_ref, degb_ref, b2_ref, o_ref):
    dinv, dinv2 = _dinvs(dega_ref[...], degb_ref[...])
    o = dinv * (ga_ref[...] + gb_ref[...]) + dinv2 * v_ref[...] + b2_ref[...]
    col = lax.broadcasted_iota(jnp.int32, o.shape, 1)
    m = col < 6
    neg = jnp.float32(-1e30)
    mx = jnp.max(jnp.where(m, o, neg), axis=1, keepdims=True)
    ex = jnp.where(m, jnp.exp(o - mx), 0.0)
    se = jnp.sum(ex, axis=1, keepdims=True)
    o_ref[...] = (o - mx) - jnp.log(se)


def _rows(F):
    return pl.BlockSpec((_R, F), lambda i: (i, 0))


def _full(shape):
    return pl.BlockSpec(shape, lambda i: tuple(0 for _ in shape))


_f32 = jnp.float32


def _tc_mm1(x, w1, dega, degb):
    return pl.pallas_call(
        _tc_mm1_body,
        grid=(_G,),
        in_specs=[_rows(D_IN), _full((D_IN, D_HID)), _rows(16), _rows(16)],
        out_specs=[_rows(D_HID), _rows(D_HID)],
        out_shape=[jax.ShapeDtypeStruct((NP, D_HID), _f32)] * 2,
    )(x, w1, dega, degb)


def _tc_comb(ga, gb, v, dega, degb):
    F = ga.shape[1]
    return pl.pallas_call(
        _tc_comb_body,
        grid=(_G,),
        in_specs=[_rows(F), _rows(F), _rows(F), _rows(16), _rows(16)],
        out_specs=[_rows(F), _rows(F)],
        out_shape=[jax.ShapeDtypeStruct((NP, F), _f32)] * 2,
    )(ga, gb, v, dega, degb)


def _tc_mid(ga, gb, v, dega, degb, b1r, w2p):
    return pl.pallas_call(
        _tc_mid_body,
        grid=(_G,),
        in_specs=[_rows(D_HID), _rows(D_HID), _rows(D_HID), _rows(16),
                  _rows(16), _full((1, D_HID)), _full((D_HID, D_PAD))],
        out_specs=[_rows(D_PAD), _rows(D_PAD)],
        out_shape=[jax.ShapeDtypeStruct((NP, D_PAD), _f32)] * 2,
    )(ga, gb, v, dega, degb, b1r, w2p)


def _tc_out(ga, gb, v, dega, degb, b2r):
    return pl.pallas_call(
        _tc_out_body,
        grid=(_G,),
        in_specs=[_rows(D_PAD), _rows(D_PAD), _rows(D_PAD), _rows(16),
                  _rows(16), _full((1, D_PAD))],
        out_specs=_rows(D_PAD),
        out_shape=jax.ShapeDtypeStruct((NP, D_PAD), _f32),
    )(ga, gb, v, dega, degb, b2r)


def kernel(x, edge_index, W1, b1, W2, b2):
    xp = jnp.pad(x, ((0, NP - N), (0, 0)))
    src2d = edge_index[0].reshape(NW, NBLK, BB)
    dst2d = edge_index[1].reshape(NW, NBLK, BB)
    b1r = b1.reshape(1, D_HID)
    w2p = jnp.zeros((D_HID, D_PAD), _f32).at[:, :6].set(W2)
    b2r = jnp.zeros((1, D_PAD), _f32).at[0, :6].set(b2)

    dega, degb = _deg(dst2d)
    t1, u1 = _tc_mm1(xp, W1, dega, degb)
    ga, gb = _prop64(u1, src2d, dst2d)
    x1, u2 = _tc_comb(ga, gb, t1, dega, degb)
    ga, gb = _prop64(u2, src2d, dst2d)
    t2, u3 = _tc_mid(ga, gb, x1, dega, degb, b1r, w2p)
    ga, gb = _prop16(u3, src2d, dst2d)
    x3, u4 = _tc_comb(ga, gb, t2, dega, degb)
    ga, gb = _prop16(u4, src2d, dst2d)
    o = _tc_out(ga, gb, x3, dega, degb, b2r)
    return o[:N, :6]
